# baseline (speedup 1.0000x reference)
import functools

import numpy as np
import jax
import jax.numpy as jnp
from jax import lax
from jax.experimental import pallas as pl
from jax.experimental.pallas import tpu as pltpu
from jax.experimental.pallas import tpu_sc as plsc

K = 8192
D = 256
B = 4096
MU = 0.25
_Z = np.int32(0)

BT = 512
KT = 2048
NB = B // BT
NK = K // KT


def _level_body(rprev_ref, qprev_ref, w_ref, wnt_ref, idx_ref, r_ref, n_ref,
                bval_ref, bidx_ref):
    k = pl.program_id(1)
    r = rprev_ref[...] - qprev_ref[...]
    rn = jnp.sum(r * r, axis=1, keepdims=True)

    @pl.when(k == 0)
    def _init():
        r_ref[...] = r
        n_ref[...] = rn
        bval_ref[...] = jnp.full((BT, 1), jnp.inf, jnp.float32)
        bidx_ref[...] = jnp.zeros((BT, 1), jnp.int32)

    w = w_ref[...]
    mm2 = lax.dot_general(r * (-2.0), w, (((1,), (1,)), ((), ())))
    d2 = (rn + wnt_ref[...]) + mm2

    tmin = jnp.min(d2, axis=1, keepdims=True)
    kio = lax.broadcasted_iota(jnp.int32, (BT, KT), 1)
    cand = jnp.where(d2 == tmin, kio, K)
    targ = jnp.min(cand, axis=1, keepdims=True) + k * KT

    upd = tmin < bval_ref[...]
    bidx_ref[...] = jnp.where(upd, targ, bidx_ref[...])
    bval_ref[...] = jnp.where(upd, tmin, bval_ref[...])

    @pl.when(k == pl.num_programs(1) - 1)
    def _fin():
        idx_ref[...] = bidx_ref[...]


_level = pl.pallas_call(
    _level_body,
    grid=(NB, NK),
    in_specs=[
        pl.BlockSpec((BT, D), lambda b, k: (b, _Z)),
        pl.BlockSpec((BT, D), lambda b, k: (b, _Z)),
        pl.BlockSpec((KT, D), lambda b, k: (k, _Z)),
        pl.BlockSpec((1, KT), lambda b, k: (_Z, k)),
    ],
    out_specs=[
        pl.BlockSpec((BT, 1), lambda b, k: (b, _Z)),
        pl.BlockSpec((BT, D), lambda b, k: (b, _Z)),
        pl.BlockSpec((BT, 1), lambda b, k: (b, _Z)),
    ],
    out_shape=[
        jax.ShapeDtypeStruct((B, 1), jnp.int32),
        jax.ShapeDtypeStruct((B, D), jnp.float32),
        jax.ShapeDtypeStruct((B, 1), jnp.float32),
    ],
    scratch_shapes=[
        pltpu.VMEM((BT, 1), jnp.float32),
        pltpu.VMEM((BT, 1), jnp.int32),
    ],
)


KW = 2048


def _wnorm_body(w_ref, out_ref):
    w = w_ref[...]
    out_ref[...] = jnp.sum(w * w, axis=1).reshape(1, 1, KW)


_wnorm = pl.pallas_call(
    _wnorm_body,
    grid=(3, K // KW),
    in_specs=[
        pl.BlockSpec((KW, D), lambda l, k: (l * (K // KW) + k, _Z)),
    ],
    out_specs=pl.BlockSpec((1, 1, KW), lambda l, k: (l, _Z, k)),
    out_shape=jax.ShapeDtypeStruct((3, 1, K), jnp.float32),
)


@functools.lru_cache(maxsize=1)
def _make_sc_gather():
    info = plsc.get_sparse_core_info()
    nc = info.num_cores
    nw = nc * info.num_subcores
    bpw = B // nw

    @functools.partial(
        pl.kernel,
        out_type=jax.ShapeDtypeStruct((B, D), jnp.float32),
        mesh=plsc.VectorSubcoreMesh(core_axis_name="c", subcore_axis_name="s"),
        scratch_types=[
            pltpu.VMEM((bpw,), jnp.int32),
            pltpu.VMEM((bpw, D), jnp.float32),
            pltpu.SemaphoreType.DMA,
        ],
    )
    def _sc_gather_kernel(table_hbm, idx_hbm, out_hbm, idx_v, rows_v, sem):
        wid = lax.axis_index("s") * nc + lax.axis_index("c")
        base = wid * bpw
        pltpu.sync_copy(idx_hbm.at[pl.ds(base, bpw)], idx_v)
        pltpu.async_copy(table_hbm.at[idx_v], rows_v, sem).wait()
        pltpu.sync_copy(rows_v, out_hbm.at[pl.ds(base, bpw)])

    return _sc_gather_kernel


def _gather(table, idx):
    return _make_sc_gather()(table, idx)


def _finale_body(x_ref, r2_ref, q2_ref, w3_ref, n1_ref, n2_ref,
                 quant_ref, loss_ref):
    r3 = r2_ref[...] - q2_ref[...]
    q3 = w3_ref[...]
    d3 = q3 - r3
    quant_ref[...] = (x_ref[...] - r3) + q3
    n3 = jnp.sum(r3 * r3, axis=1, keepdims=True)
    nq = jnp.sum(d3 * d3, axis=1, keepdims=True)
    c = jnp.float32((1.0 + MU) / D)
    loss_ref[...] = ((n1_ref[...] * c + n2_ref[...] * c) + n3 * c) + nq * c


_finale = pl.pallas_call(
    _finale_body,
    grid=(NB,),
    in_specs=[
        pl.BlockSpec((BT, D), lambda b: (b, _Z)),
        pl.BlockSpec((BT, D), lambda b: (b, _Z)),
        pl.BlockSpec((BT, D), lambda b: (b, _Z)),
        pl.BlockSpec((1, D), lambda b: (_Z, _Z)),
        pl.BlockSpec((BT, 1), lambda b: (b, _Z)),
        pl.BlockSpec((BT, 1), lambda b: (b, _Z)),
    ],
    out_specs=[
        pl.BlockSpec((BT, D), lambda b: (b, _Z)),
        pl.BlockSpec((BT, 1), lambda b: (b, _Z)),
    ],
    out_shape=[
        jax.ShapeDtypeStruct((B, D), jnp.float32),
        jax.ShapeDtypeStruct((B, 1), jnp.float32),
    ],
)


def kernel(x, codebooks):
    x = x.astype(jnp.float32)
    codebooks = codebooks.astype(jnp.float32)
    w0 = codebooks[0]
    w1 = codebooks[1]
    w2 = codebooks[2]
    w3row = codebooks[3, 0:1, :]

    zeros = jnp.zeros_like(x)
    wn = _wnorm(codebooks[0:3].reshape(3 * K, D))
    idx0, r0, _ = _level(x, zeros, w0, wn[0])
    q0 = _gather(w0, idx0.reshape(B))
    idx1, r1, n1 = _level(r0, q0, w1, wn[1])
    q1 = _gather(w1, idx1.reshape(B))
    idx2, r2, n2 = _level(r1, q1, w2, wn[2])
    q2 = _gather(w2, idx2.reshape(B))
    quant, loss = _finale(x, r2, q2, w3row, n1, n2)

    zcol = jnp.zeros((B, 1), jnp.int32)
    indices = jnp.concatenate([idx0, idx1, idx2, zcol], axis=1)
    return quant, indices.astype(jnp.int64), loss.reshape(B)


# speedup vs baseline: 1.0969x; 1.0969x over previous
import functools

import numpy as np
import jax
import jax.numpy as jnp
from jax import lax
from jax.experimental import pallas as pl
from jax.experimental.pallas import tpu as pltpu
from jax.experimental.pallas import tpu_sc as plsc

K = 8192
D = 256
B = 4096
MU = 0.25
_Z = np.int32(0)

BT = 512
KT = 2048
NB = B // BT
NK = K // KT


def _make_level(lvl, has_q):
    def body(*refs):
        if has_q:
            (rprev_ref, qprev_ref, cb_ref, idx_ref, gidx_ref, r_ref, n_ref,
             wnt_s, bval_ref, bidx_ref) = refs
        else:
            (rprev_ref, cb_ref, idx_ref, gidx_ref, r_ref, n_ref,
             wnt_s, bval_ref, bidx_ref) = refs
        b = pl.program_id(0)
        k = pl.program_id(1)
        if has_q:
            r = rprev_ref[...] - qprev_ref[...]
        else:
            r = rprev_ref[...]
        rn = jnp.sum(r * r, axis=1, keepdims=True)

        @pl.when(k == 0)
        def _init():
            r_ref[...] = r
            n_ref[...] = rn
            bval_ref[...] = jnp.full((BT, 1), jnp.inf, jnp.float32)
            bidx_ref[...] = jnp.zeros((BT, 1), jnp.int32)

        w = cb_ref[...]

        @pl.when(b == 0)
        def _norms():
            wnt_s[pl.ds(k, 1), :] = jnp.sum(w * w, axis=1).reshape(1, KT)

        wnt = wnt_s[pl.ds(k, 1), :]
        mm2 = lax.dot_general(r * (-2.0), w, (((1,), (1,)), ((), ())))
        d2 = (rn + wnt) + mm2

        tmin = jnp.min(d2, axis=1, keepdims=True)
        kio = lax.broadcasted_iota(jnp.int32, (BT, KT), 1)
        cand = jnp.where(d2 == tmin, kio, K)
        targ = jnp.min(cand, axis=1, keepdims=True) + k * KT

        upd = tmin < bval_ref[...]
        bidx_ref[...] = jnp.where(upd, targ, bidx_ref[...])
        bval_ref[...] = jnp.where(upd, tmin, bval_ref[...])

        @pl.when(k == pl.num_programs(1) - 1)
        def _fin():
            best = bidx_ref[...]
            idx_ref[...] = best
            gidx_ref[...] = best + np.int32(lvl * K)

    base = np.int32(lvl * NK)
    in_specs = [pl.BlockSpec((BT, D), lambda b, k: (b, _Z))]
    if has_q:
        in_specs.append(pl.BlockSpec((BT, D), lambda b, k: (b, _Z)))
    in_specs.append(pl.BlockSpec((KT, D), lambda b, k: (base + k, _Z)))
    return pl.pallas_call(
        body,
        grid=(NB, NK),
        in_specs=in_specs,
        out_specs=[
            pl.BlockSpec((BT, 1), lambda b, k: (b, _Z)),
            pl.BlockSpec((BT, 1), lambda b, k: (b, _Z)),
            pl.BlockSpec((BT, D), lambda b, k: (b, _Z)),
            pl.BlockSpec((BT, 1), lambda b, k: (b, _Z)),
        ],
        out_shape=[
            jax.ShapeDtypeStruct((B, 1), jnp.int32),
            jax.ShapeDtypeStruct((B, 1), jnp.int32),
            jax.ShapeDtypeStruct((B, D), jnp.float32),
            jax.ShapeDtypeStruct((B, 1), jnp.float32),
        ],
        scratch_shapes=[
            pltpu.VMEM((NK, KT), jnp.float32),
            pltpu.VMEM((BT, 1), jnp.float32),
            pltpu.VMEM((BT, 1), jnp.int32),
        ],
    )


_level0 = _make_level(0, has_q=False)
_level1 = _make_level(1, has_q=True)
_level2 = _make_level(2, has_q=True)


@functools.lru_cache(maxsize=1)
def _make_sc_gather():
    info = plsc.get_sparse_core_info()
    nc = info.num_cores
    nw = nc * info.num_subcores
    bpw = B // nw

    @functools.partial(
        pl.kernel,
        out_type=jax.ShapeDtypeStruct((B, D), jnp.float32),
        mesh=plsc.VectorSubcoreMesh(core_axis_name="c", subcore_axis_name="s"),
        scratch_types=[
            pltpu.VMEM((bpw,), jnp.int32),
            pltpu.VMEM((bpw, D), jnp.float32),
            pltpu.SemaphoreType.DMA,
        ],
    )
    def _sc_gather_kernel(table_hbm, idx_hbm, out_hbm, idx_v, rows_v, sem):
        wid = lax.axis_index("s") * nc + lax.axis_index("c")
        base = wid * bpw
        pltpu.sync_copy(idx_hbm.at[pl.ds(base, bpw)], idx_v)
        pltpu.async_copy(table_hbm.at[idx_v], rows_v, sem).wait()
        pltpu.sync_copy(rows_v, out_hbm.at[pl.ds(base, bpw)])

    return _sc_gather_kernel


def _gather(table, idx):
    return _make_sc_gather()(table, idx)


def _finale_body(x_ref, r2_ref, q2_ref, w3_ref, n1_ref, n2_ref,
                 i0_ref, i1_ref, i2_ref, quant_ref, loss_ref, ipack_ref):
    r3 = r2_ref[...] - q2_ref[...]
    q3 = w3_ref[...]
    d3 = q3 - r3
    quant_ref[...] = (x_ref[...] - r3) + q3
    n3 = jnp.sum(r3 * r3, axis=1, keepdims=True)
    nq = jnp.sum(d3 * d3, axis=1, keepdims=True)
    c = jnp.float32((1.0 + MU) / D)
    loss_ref[...] = ((n1_ref[...] * c + n2_ref[...] * c) + n3 * c) + nq * c
    zc = jnp.zeros((BT, 1), jnp.int32)
    ipack_ref[...] = jnp.concatenate(
        [i0_ref[...], zc, i1_ref[...], zc, i2_ref[...], zc, zc, zc], axis=1)


_finale = pl.pallas_call(
    _finale_body,
    grid=(NB,),
    in_specs=[
        pl.BlockSpec((BT, D), lambda b: (b, _Z)),
        pl.BlockSpec((BT, D), lambda b: (b, _Z)),
        pl.BlockSpec((BT, D), lambda b: (b, _Z)),
        pl.BlockSpec((1, D), lambda b: (_Z, _Z)),
        pl.BlockSpec((BT, 1), lambda b: (b, _Z)),
        pl.BlockSpec((BT, 1), lambda b: (b, _Z)),
        pl.BlockSpec((BT, 1), lambda b: (b, _Z)),
        pl.BlockSpec((BT, 1), lambda b: (b, _Z)),
        pl.BlockSpec((BT, 1), lambda b: (b, _Z)),
    ],
    out_specs=[
        pl.BlockSpec((BT, D), lambda b: (b, _Z)),
        pl.BlockSpec((BT, 1), lambda b: (b, _Z)),
        pl.BlockSpec((BT, 8), lambda b: (b, _Z)),
    ],
    out_shape=[
        jax.ShapeDtypeStruct((B, D), jnp.float32),
        jax.ShapeDtypeStruct((B, 1), jnp.float32),
        jax.ShapeDtypeStruct((B, 8), jnp.int32),
    ],
)


def kernel(x, codebooks):
    x = x.astype(jnp.float32)
    codebooks = codebooks.astype(jnp.float32)
    cb = codebooks.reshape(4 * K, D)
    w3row = codebooks[3, 0:1, :]

    idx0, g0, r0, _ = _level0(x, cb)
    q0 = _gather(cb, g0.reshape(B))
    idx1, g1, r1, n1 = _level1(r0, q0, cb)
    q1 = _gather(cb, g1.reshape(B))
    idx2, g2, r2, n2 = _level2(r1, q1, cb)
    q2 = _gather(cb, g2.reshape(B))
    quant, loss, ipack = _finale(x, r2, q2, w3row, n1, n2, idx0, idx1, idx2)

    indices = lax.bitcast_convert_type(ipack.reshape(B, 4, 2), jnp.int64)
    return quant, indices, loss.reshape(B)


# speedup vs baseline: 1.2351x; 1.1259x over previous
import functools

import numpy as np
import jax
import jax.numpy as jnp
from jax import lax
from jax.experimental import pallas as pl
from jax.experimental.pallas import tpu as pltpu
from jax.experimental.pallas import tpu_sc as plsc

K = 8192
D = 256
B = 4096
MU = 0.25
_Z = np.int32(0)

BT = 512
KT = 2048
NB = B // BT
NK = K // KT


def _make_level(lvl, has_q):
    def body(*refs):
        if has_q:
            (rprev_ref, qprev_ref, cb_ref, idx_ref, gidx_ref, r_ref, n_ref,
             bval_ref, bidx_ref) = refs
        else:
            (rprev_ref, cb_ref, idx_ref, gidx_ref, r_ref, n_ref,
             bval_ref, bidx_ref) = refs
        k = pl.program_id(1)
        if has_q:
            r = rprev_ref[...] - qprev_ref[...]
        else:
            r = rprev_ref[...]
        rn = jnp.sum(r * r, axis=1, keepdims=True)

        @pl.when(k == 0)
        def _init():
            r_ref[...] = r
            n_ref[...] = rn
            bval_ref[...] = jnp.full((BT, 1), jnp.inf, jnp.float32)
            bidx_ref[...] = jnp.zeros((BT, 1), jnp.int32)

        w = cb_ref[...]
        wnt = jnp.sum(w * w, axis=1).reshape(1, KT)
        mm2 = lax.dot_general(r * (-2.0), w, (((1,), (1,)), ((), ())))
        d2 = (rn + wnt) + mm2

        tmin = jnp.min(d2, axis=1, keepdims=True)
        kio = lax.broadcasted_iota(jnp.int32, (BT, KT), 1)
        cand = jnp.where(d2 == tmin, kio, K)
        targ = jnp.min(cand, axis=1, keepdims=True) + k * KT

        upd = tmin < bval_ref[...]
        bidx_ref[...] = jnp.where(upd, targ, bidx_ref[...])
        bval_ref[...] = jnp.where(upd, tmin, bval_ref[...])

        @pl.when(k == pl.num_programs(1) - 1)
        def _fin():
            best = bidx_ref[...]
            idx_ref[...] = best
            gidx_ref[...] = best + np.int32(lvl * K)

    base = np.int32(lvl * NK)
    in_specs = [pl.BlockSpec((BT, D), lambda b, k: (b, _Z))]
    if has_q:
        in_specs.append(pl.BlockSpec((BT, D), lambda b, k: (b, _Z)))
    in_specs.append(pl.BlockSpec((KT, D), lambda b, k: (base + k, _Z)))
    return pl.pallas_call(
        body,
        grid=(NB, NK),
        in_specs=in_specs,
        out_specs=[
            pl.BlockSpec((BT, 1), lambda b, k: (b, _Z)),
            pl.BlockSpec((BT, 1), lambda b, k: (b, _Z)),
            pl.BlockSpec((BT, D), lambda b, k: (b, _Z)),
            pl.BlockSpec((BT, 1), lambda b, k: (b, _Z)),
        ],
        out_shape=[
            jax.ShapeDtypeStruct((B, 1), jnp.int32),
            jax.ShapeDtypeStruct((B, 1), jnp.int32),
            jax.ShapeDtypeStruct((B, D), jnp.float32),
            jax.ShapeDtypeStruct((B, 1), jnp.float32),
        ],
        scratch_shapes=[
            pltpu.VMEM((BT, 1), jnp.float32),
            pltpu.VMEM((BT, 1), jnp.int32),
        ],
    )


_level0 = _make_level(0, has_q=False)
_level1 = _make_level(1, has_q=True)
_level2 = _make_level(2, has_q=True)


@functools.lru_cache(maxsize=1)
def _make_sc_gather():
    info = plsc.get_sparse_core_info()
    nc = info.num_cores
    nw = nc * info.num_subcores
    bpw = B // nw

    @functools.partial(
        pl.kernel,
        out_type=jax.ShapeDtypeStruct((B, D), jnp.float32),
        mesh=plsc.VectorSubcoreMesh(core_axis_name="c", subcore_axis_name="s"),
        scratch_types=[
            pltpu.VMEM((bpw,), jnp.int32),
            pltpu.VMEM((bpw, D), jnp.float32),
            pltpu.SemaphoreType.DMA,
        ],
    )
    def _sc_gather_kernel(table_hbm, idx_hbm, out_hbm, idx_v, rows_v, sem):
        wid = lax.axis_index("s") * nc + lax.axis_index("c")
        base = wid * bpw
        pltpu.sync_copy(idx_hbm.at[pl.ds(base, bpw)], idx_v)
        pltpu.async_copy(table_hbm.at[idx_v], rows_v, sem).wait()
        pltpu.sync_copy(rows_v, out_hbm.at[pl.ds(base, bpw)])

    return _sc_gather_kernel


def _gather(table, idx):
    return _make_sc_gather()(table, idx)


def _finale_body(x_ref, r2_ref, q2_ref, w3_ref, n1_ref, n2_ref,
                 i0_ref, i1_ref, i2_ref, quant_ref, loss_ref, ipack_ref):
    r3 = r2_ref[...] - q2_ref[...]
    q3 = w3_ref[...]
    d3 = q3 - r3
    quant_ref[...] = (x_ref[...] - r3) + q3
    n3 = jnp.sum(r3 * r3, axis=1, keepdims=True)
    nq = jnp.sum(d3 * d3, axis=1, keepdims=True)
    c = jnp.float32((1.0 + MU) / D)
    loss_ref[...] = ((n1_ref[...] * c + n2_ref[...] * c) + n3 * c) + nq * c
    zc = jnp.zeros((BT, 1), jnp.int32)
    ipack_ref[...] = jnp.concatenate(
        [i0_ref[...], zc, i1_ref[...], zc, i2_ref[...], zc, zc, zc], axis=1)


_finale = pl.pallas_call(
    _finale_body,
    grid=(NB,),
    in_specs=[
        pl.BlockSpec((BT, D), lambda b: (b, _Z)),
        pl.BlockSpec((BT, D), lambda b: (b, _Z)),
        pl.BlockSpec((BT, D), lambda b: (b, _Z)),
        pl.BlockSpec((1, D), lambda b: (_Z, _Z)),
        pl.BlockSpec((BT, 1), lambda b: (b, _Z)),
        pl.BlockSpec((BT, 1), lambda b: (b, _Z)),
        pl.BlockSpec((BT, 1), lambda b: (b, _Z)),
        pl.BlockSpec((BT, 1), lambda b: (b, _Z)),
        pl.BlockSpec((BT, 1), lambda b: (b, _Z)),
    ],
    out_specs=[
        pl.BlockSpec((BT, D), lambda b: (b, _Z)),
        pl.BlockSpec((BT, 1), lambda b: (b, _Z)),
        pl.BlockSpec((BT, 8), lambda b: (b, _Z)),
    ],
    out_shape=[
        jax.ShapeDtypeStruct((B, D), jnp.float32),
        jax.ShapeDtypeStruct((B, 1), jnp.float32),
        jax.ShapeDtypeStruct((B, 8), jnp.int32),
    ],
)


def kernel(x, codebooks):
    x = x.astype(jnp.float32)
    codebooks = codebooks.astype(jnp.float32)
    cb = codebooks.reshape(4 * K, D)
    w3row = codebooks[3, 0:1, :]

    idx0, g0, r0, _ = _level0(x, cb)
    q0 = _gather(cb, g0.reshape(B))
    idx1, g1, r1, n1 = _level1(r0, q0, cb)
    q1 = _gather(cb, g1.reshape(B))
    idx2, g2, r2, n2 = _level2(r1, q1, cb)
    q2 = _gather(cb, g2.reshape(B))
    quant, loss, ipack = _finale(x, r2, q2, w3row, n1, n2, idx0, idx1, idx2)

    indices = lax.bitcast_convert_type(ipack.reshape(B, 4, 2), jnp.int64)
    return quant, indices, loss.reshape(B)


# speedup vs baseline: 1.2631x; 1.0227x over previous
import functools

import numpy as np
import jax
import jax.numpy as jnp
from jax import lax
from jax.experimental import pallas as pl
from jax.experimental.pallas import tpu as pltpu
from jax.experimental.pallas import tpu_sc as plsc

K = 8192
D = 256
B = 4096
MU = 0.25
_Z = np.int32(0)

BT = 512
KT = 2048
NB = B // BT
NK = K // KT


def _make_level(lvl, has_q):
    def body(*refs):
        if has_q:
            (rprev_ref, qprev_ref, cb_ref, idx_ref, gidx_ref, r_ref, n_ref,
             bval_ref, bidx_ref) = refs
        else:
            (rprev_ref, cb_ref, idx_ref, gidx_ref, r_ref, n_ref,
             bval_ref, bidx_ref) = refs
        k = pl.program_id(1)
        if has_q:
            r = rprev_ref[...] - qprev_ref[...]
        else:
            r = rprev_ref[...]
        rn = jnp.sum(r * r, axis=1, keepdims=True)

        @pl.when(k == 0)
        def _init():
            r_ref[...] = r
            n_ref[...] = rn
            bval_ref[...] = jnp.full((BT, 1), jnp.inf, jnp.float32)
            bidx_ref[...] = jnp.zeros((BT, 1), jnp.int32)

        w = cb_ref[...]
        wnt = jnp.sum(w * w, axis=1).reshape(1, KT)
        mm2 = lax.dot_general(r * (-2.0), w, (((1,), (1,)), ((), ())))
        d2 = (rn + wnt) + mm2

        tmin = jnp.min(d2, axis=1, keepdims=True)
        kio = lax.broadcasted_iota(jnp.int32, (BT, KT), 1)
        cand = jnp.where(d2 == tmin, kio, K)
        targ = jnp.min(cand, axis=1, keepdims=True) + k * KT

        upd = tmin < bval_ref[...]
        bidx_ref[...] = jnp.where(upd, targ, bidx_ref[...])
        bval_ref[...] = jnp.where(upd, tmin, bval_ref[...])

        @pl.when(k == pl.num_programs(1) - 1)
        def _fin():
            best = bidx_ref[...]
            idx_ref[...] = best
            gidx_ref[...] = (best + np.int32(lvl * K)).reshape(1, BT)

    base = np.int32(lvl * NK)
    in_specs = [pl.BlockSpec((BT, D), lambda b, k: (b, _Z))]
    if has_q:
        in_specs.append(pl.BlockSpec((BT, D), lambda b, k: (b, _Z)))
    in_specs.append(pl.BlockSpec((KT, D), lambda b, k: (base + k, _Z)))
    return pl.pallas_call(
        body,
        grid=(NB, NK),
        in_specs=in_specs,
        out_specs=[
            pl.BlockSpec((BT, 1), lambda b, k: (b, _Z)),
            pl.BlockSpec((1, BT), lambda b, k: (_Z, b)),
            pl.BlockSpec((BT, D), lambda b, k: (b, _Z)),
            pl.BlockSpec((BT, 1), lambda b, k: (b, _Z)),
        ],
        out_shape=[
            jax.ShapeDtypeStruct((B, 1), jnp.int32),
            jax.ShapeDtypeStruct((1, B), jnp.int32),
            jax.ShapeDtypeStruct((B, D), jnp.float32),
            jax.ShapeDtypeStruct((B, 1), jnp.float32),
        ],
        scratch_shapes=[
            pltpu.VMEM((BT, 1), jnp.float32),
            pltpu.VMEM((BT, 1), jnp.int32),
        ],
    )


_level0 = _make_level(0, has_q=False)
_level1 = _make_level(1, has_q=True)
_level2 = _make_level(2, has_q=True)


@functools.lru_cache(maxsize=1)
def _make_sc_gather():
    info = plsc.get_sparse_core_info()
    nc = info.num_cores
    nw = nc * info.num_subcores
    bpw = B // nw

    @functools.partial(
        pl.kernel,
        out_type=jax.ShapeDtypeStruct((B, D), jnp.float32),
        mesh=plsc.VectorSubcoreMesh(core_axis_name="c", subcore_axis_name="s"),
        scratch_types=[
            pltpu.VMEM((bpw,), jnp.int32),
            pltpu.VMEM((bpw, D), jnp.float32),
            pltpu.SemaphoreType.DMA,
        ],
    )
    def _sc_gather_kernel(table_hbm, idx_hbm, out_hbm, idx_v, rows_v, sem):
        wid = lax.axis_index("s") * nc + lax.axis_index("c")
        base = wid * bpw
        pltpu.sync_copy(idx_hbm.at[pl.ds(base, bpw)], idx_v)
        pltpu.async_copy(table_hbm.at[idx_v], rows_v, sem).wait()
        pltpu.sync_copy(rows_v, out_hbm.at[pl.ds(base, bpw)])

    return _sc_gather_kernel


def _gather(table, idx):
    return _make_sc_gather()(table, idx)


def _finale_body(x_ref, r2_ref, q2_ref, w3_ref, n1_ref, n2_ref,
                 i0_ref, i1_ref, i2_ref, quant_ref, loss_ref, ipack_ref):
    r3 = r2_ref[...] - q2_ref[...]
    q3 = w3_ref[...]
    d3 = q3 - r3
    quant_ref[...] = (x_ref[...] - r3) + q3
    n3 = jnp.sum(r3 * r3, axis=1, keepdims=True)
    nq = jnp.sum(d3 * d3, axis=1, keepdims=True)
    c = jnp.float32((1.0 + MU) / D)
    loss_ref[...] = ((n1_ref[...] * c + n2_ref[...] * c) + n3 * c) + nq * c
    zc = jnp.zeros((BT, 1), jnp.int32)
    ipack_ref[...] = jnp.concatenate(
        [i0_ref[...], zc, i1_ref[...], zc, i2_ref[...], zc, zc, zc], axis=1)


_finale = pl.pallas_call(
    _finale_body,
    grid=(NB,),
    in_specs=[
        pl.BlockSpec((BT, D), lambda b: (b, _Z)),
        pl.BlockSpec((BT, D), lambda b: (b, _Z)),
        pl.BlockSpec((BT, D), lambda b: (b, _Z)),
        pl.BlockSpec((1, D), lambda b: (_Z, _Z)),
        pl.BlockSpec((BT, 1), lambda b: (b, _Z)),
        pl.BlockSpec((BT, 1), lambda b: (b, _Z)),
        pl.BlockSpec((BT, 1), lambda b: (b, _Z)),
        pl.BlockSpec((BT, 1), lambda b: (b, _Z)),
        pl.BlockSpec((BT, 1), lambda b: (b, _Z)),
    ],
    out_specs=[
        pl.BlockSpec((BT, D), lambda b: (b, _Z)),
        pl.BlockSpec((BT, 1), lambda b: (b, _Z)),
        pl.BlockSpec((BT, 8), lambda b: (b, _Z)),
    ],
    out_shape=[
        jax.ShapeDtypeStruct((B, D), jnp.float32),
        jax.ShapeDtypeStruct((B, 1), jnp.float32),
        jax.ShapeDtypeStruct((B, 8), jnp.int32),
    ],
)


def kernel(x, codebooks):
    x = x.astype(jnp.float32)
    codebooks = codebooks.astype(jnp.float32)
    cb = codebooks.reshape(4 * K, D)
    w3row = codebooks[3, 0:1, :]

    idx0, g0, r0, _ = _level0(x, cb)
    q0 = _gather(cb, g0.reshape(B))
    idx1, g1, r1, n1 = _level1(r0, q0, cb)
    q1 = _gather(cb, g1.reshape(B))
    idx2, g2, r2, n2 = _level2(r1, q1, cb)
    q2 = _gather(cb, g2.reshape(B))
    quant, loss, ipack = _finale(x, r2, q2, w3row, n1, n2, idx0, idx1, idx2)

    indices = lax.bitcast_convert_type(ipack.reshape(B, 4, 2), jnp.int64)
    return quant, indices, loss.reshape(B)


# speedup vs baseline: 1.3648x; 1.0805x over previous
import functools

import numpy as np
import jax
import jax.numpy as jnp
from jax import lax
from jax.experimental import pallas as pl
from jax.experimental.pallas import tpu as pltpu
from jax.experimental.pallas import tpu_sc as plsc

K = 8192
D = 256
B = 4096
MU = 0.25
_Z = np.int32(0)

BT = 1024
KT = 2048
NB = B // BT
NK = K // KT


def _make_level(lvl, has_q):
    def body(*refs):
        if has_q:
            (rprev_ref, qprev_ref, cb_ref, idx_ref, gidx_ref, r_ref, n_ref,
             bval_ref, bidx_ref) = refs
        else:
            (rprev_ref, cb_ref, idx_ref, gidx_ref, r_ref, n_ref,
             bval_ref, bidx_ref) = refs
        k = pl.program_id(1)
        if has_q:
            r = rprev_ref[...] - qprev_ref[...]
        else:
            r = rprev_ref[...]
        rn = jnp.sum(r * r, axis=1, keepdims=True)

        @pl.when(k == 0)
        def _init():
            r_ref[...] = r
            n_ref[...] = rn
            bval_ref[...] = jnp.full((BT, 1), jnp.inf, jnp.float32)
            bidx_ref[...] = jnp.zeros((BT, 1), jnp.int32)

        w = cb_ref[...]
        wnt = jnp.sum(w * w, axis=1).reshape(1, KT)
        mm2 = lax.dot_general(r * (-2.0), w, (((1,), (1,)), ((), ())))
        d2 = (rn + wnt) + mm2

        tmin = jnp.min(d2, axis=1, keepdims=True)
        kio = lax.broadcasted_iota(jnp.int32, (BT, KT), 1)
        cand = jnp.where(d2 == tmin, kio, K)
        targ = jnp.min(cand, axis=1, keepdims=True) + k * KT

        upd = tmin < bval_ref[...]
        bidx_ref[...] = jnp.where(upd, targ, bidx_ref[...])
        bval_ref[...] = jnp.where(upd, tmin, bval_ref[...])

        @pl.when(k == pl.num_programs(1) - 1)
        def _fin():
            best = bidx_ref[...]
            idx_ref[...] = best
            gidx_ref[...] = (best + np.int32(lvl * K)).reshape(1, BT)

    base = np.int32(lvl * NK)
    in_specs = [pl.BlockSpec((BT, D), lambda b, k: (b, _Z))]
    if has_q:
        in_specs.append(pl.BlockSpec((BT, D), lambda b, k: (b, _Z)))
    in_specs.append(pl.BlockSpec((KT, D), lambda b, k: (base + k, _Z)))
    return pl.pallas_call(
        body,
        grid=(NB, NK),
        in_specs=in_specs,
        out_specs=[
            pl.BlockSpec((BT, 1), lambda b, k: (b, _Z)),
            pl.BlockSpec((1, BT), lambda b, k: (_Z, b)),
            pl.BlockSpec((BT, D), lambda b, k: (b, _Z)),
            pl.BlockSpec((BT, 1), lambda b, k: (b, _Z)),
        ],
        out_shape=[
            jax.ShapeDtypeStruct((B, 1), jnp.int32),
            jax.ShapeDtypeStruct((1, B), jnp.int32),
            jax.ShapeDtypeStruct((B, D), jnp.float32),
            jax.ShapeDtypeStruct((B, 1), jnp.float32),
        ],
        scratch_shapes=[
            pltpu.VMEM((BT, 1), jnp.float32),
            pltpu.VMEM((BT, 1), jnp.int32),
        ],
    )


_level0 = _make_level(0, has_q=False)
_level1 = _make_level(1, has_q=True)
_level2 = _make_level(2, has_q=True)


@functools.lru_cache(maxsize=1)
def _make_sc_gather():
    info = plsc.get_sparse_core_info()
    nc = info.num_cores
    nw = nc * info.num_subcores
    bpw = B // nw

    @functools.partial(
        pl.kernel,
        out_type=jax.ShapeDtypeStruct((B, D), jnp.float32),
        mesh=plsc.VectorSubcoreMesh(core_axis_name="c", subcore_axis_name="s"),
        scratch_types=[
            pltpu.VMEM((bpw,), jnp.int32),
            pltpu.VMEM((bpw, D), jnp.float32),
            pltpu.SemaphoreType.DMA,
        ],
    )
    def _sc_gather_kernel(table_hbm, idx_hbm, out_hbm, idx_v, rows_v, sem):
        wid = lax.axis_index("s") * nc + lax.axis_index("c")
        base = wid * bpw
        pltpu.sync_copy(idx_hbm.at[pl.ds(base, bpw)], idx_v)
        pltpu.async_copy(table_hbm.at[idx_v], rows_v, sem).wait()
        pltpu.sync_copy(rows_v, out_hbm.at[pl.ds(base, bpw)])

    return _sc_gather_kernel


def _gather(table, idx):
    return _make_sc_gather()(table, idx)


def _finale_body(x_ref, r2_ref, q2_ref, w3_ref, n1_ref, n2_ref,
                 i0_ref, i1_ref, i2_ref, quant_ref, loss_ref, ipack_ref):
    r3 = r2_ref[...] - q2_ref[...]
    q3 = w3_ref[...]
    d3 = q3 - r3
    quant_ref[...] = (x_ref[...] - r3) + q3
    n3 = jnp.sum(r3 * r3, axis=1, keepdims=True)
    nq = jnp.sum(d3 * d3, axis=1, keepdims=True)
    c = jnp.float32((1.0 + MU) / D)
    loss_ref[...] = ((n1_ref[...] * c + n2_ref[...] * c) + n3 * c) + nq * c
    zc = jnp.zeros((BT, 1), jnp.int32)
    ipack_ref[...] = jnp.concatenate(
        [i0_ref[...], zc, i1_ref[...], zc, i2_ref[...], zc, zc, zc], axis=1)


_finale = pl.pallas_call(
    _finale_body,
    grid=(NB,),
    in_specs=[
        pl.BlockSpec((BT, D), lambda b: (b, _Z)),
        pl.BlockSpec((BT, D), lambda b: (b, _Z)),
        pl.BlockSpec((BT, D), lambda b: (b, _Z)),
        pl.BlockSpec((1, D), lambda b: (_Z, _Z)),
        pl.BlockSpec((BT, 1), lambda b: (b, _Z)),
        pl.BlockSpec((BT, 1), lambda b: (b, _Z)),
        pl.BlockSpec((BT, 1), lambda b: (b, _Z)),
        pl.BlockSpec((BT, 1), lambda b: (b, _Z)),
        pl.BlockSpec((BT, 1), lambda b: (b, _Z)),
    ],
    out_specs=[
        pl.BlockSpec((BT, D), lambda b: (b, _Z)),
        pl.BlockSpec((BT, 1), lambda b: (b, _Z)),
        pl.BlockSpec((BT, 8), lambda b: (b, _Z)),
    ],
    out_shape=[
        jax.ShapeDtypeStruct((B, D), jnp.float32),
        jax.ShapeDtypeStruct((B, 1), jnp.float32),
        jax.ShapeDtypeStruct((B, 8), jnp.int32),
    ],
)


def kernel(x, codebooks):
    x = x.astype(jnp.float32)
    codebooks = codebooks.astype(jnp.float32)
    cb = codebooks.reshape(4 * K, D)
    w3row = codebooks[3, 0:1, :]

    idx0, g0, r0, _ = _level0(x, cb)
    q0 = _gather(cb, g0.reshape(B))
    idx1, g1, r1, n1 = _level1(r0, q0, cb)
    q1 = _gather(cb, g1.reshape(B))
    idx2, g2, r2, n2 = _level2(r1, q1, cb)
    q2 = _gather(cb, g2.reshape(B))
    quant, loss, ipack = _finale(x, r2, q2, w3row, n1, n2, idx0, idx1, idx2)

    indices = lax.bitcast_convert_type(ipack.reshape(B, 4, 2), jnp.int64)
    return quant, indices, loss.reshape(B)


# speedup vs baseline: 1.4261x; 1.0450x over previous
import functools

import numpy as np
import jax
import jax.numpy as jnp
from jax import lax
from jax.experimental import pallas as pl
from jax.experimental.pallas import tpu as pltpu
from jax.experimental.pallas import tpu_sc as plsc

K = 8192
D = 256
B = 4096
MU = 0.25
_Z = np.int32(0)

BT = 2048
KT = 2048
NB = B // BT
NK = K // KT


def _make_level(lvl, has_q):
    def body(*refs):
        if has_q:
            (rprev_ref, qprev_ref, cb_ref, idx_ref, gidx_ref, r_ref, n_ref,
             bval_ref, bidx_ref) = refs
        else:
            (rprev_ref, cb_ref, idx_ref, gidx_ref, r_ref, n_ref,
             bval_ref, bidx_ref) = refs
        k = pl.program_id(1)
        if has_q:
            r = rprev_ref[...] - qprev_ref[...]
        else:
            r = rprev_ref[...]
        rn = jnp.sum(r * r, axis=1, keepdims=True)

        @pl.when(k == 0)
        def _init():
            r_ref[...] = r
            n_ref[...] = rn
            bval_ref[...] = jnp.full((BT, 1), jnp.inf, jnp.float32)
            bidx_ref[...] = jnp.zeros((BT, 1), jnp.int32)

        w = cb_ref[...]
        wnt = jnp.sum(w * w, axis=1).reshape(1, KT)
        mm2 = lax.dot_general(r * (-2.0), w, (((1,), (1,)), ((), ())))
        d2 = (rn + wnt) + mm2

        tmin = jnp.min(d2, axis=1, keepdims=True)
        kio = lax.broadcasted_iota(jnp.int32, (BT, KT), 1)
        cand = jnp.where(d2 == tmin, kio, K)
        targ = jnp.min(cand, axis=1, keepdims=True) + k * KT

        upd = tmin < bval_ref[...]
        bidx_ref[...] = jnp.where(upd, targ, bidx_ref[...])
        bval_ref[...] = jnp.where(upd, tmin, bval_ref[...])

        @pl.when(k == pl.num_programs(1) - 1)
        def _fin():
            best = bidx_ref[...]
            idx_ref[...] = best
            gidx_ref[...] = (best + np.int32(lvl * K)).reshape(1, BT)

    base = np.int32(lvl * NK)
    in_specs = [pl.BlockSpec((BT, D), lambda b, k: (b, _Z))]
    if has_q:
        in_specs.append(pl.BlockSpec((BT, D), lambda b, k: (b, _Z)))
    in_specs.append(pl.BlockSpec((KT, D), lambda b, k: (base + k, _Z)))
    return pl.pallas_call(
        body,
        grid=(NB, NK),
        in_specs=in_specs,
        out_specs=[
            pl.BlockSpec((BT, 1), lambda b, k: (b, _Z)),
            pl.BlockSpec((1, BT), lambda b, k: (_Z, b)),
            pl.BlockSpec((BT, D), lambda b, k: (b, _Z)),
            pl.BlockSpec((BT, 1), lambda b, k: (b, _Z)),
        ],
        out_shape=[
            jax.ShapeDtypeStruct((B, 1), jnp.int32),
            jax.ShapeDtypeStruct((1, B), jnp.int32),
            jax.ShapeDtypeStruct((B, D), jnp.float32),
            jax.ShapeDtypeStruct((B, 1), jnp.float32),
        ],
        scratch_shapes=[
            pltpu.VMEM((BT, 1), jnp.float32),
            pltpu.VMEM((BT, 1), jnp.int32),
        ],
    )


_level0 = _make_level(0, has_q=False)
_level1 = _make_level(1, has_q=True)
_level2 = _make_level(2, has_q=True)


@functools.lru_cache(maxsize=1)
def _make_sc_gather():
    info = plsc.get_sparse_core_info()
    nc = info.num_cores
    nw = nc * info.num_subcores
    bpw = B // nw

    @functools.partial(
        pl.kernel,
        out_type=jax.ShapeDtypeStruct((B, D), jnp.float32),
        mesh=plsc.VectorSubcoreMesh(core_axis_name="c", subcore_axis_name="s"),
        scratch_types=[
            pltpu.VMEM((bpw,), jnp.int32),
            pltpu.VMEM((bpw, D), jnp.float32),
            pltpu.SemaphoreType.DMA,
        ],
    )
    def _sc_gather_kernel(table_hbm, idx_hbm, out_hbm, idx_v, rows_v, sem):
        wid = lax.axis_index("s") * nc + lax.axis_index("c")
        base = wid * bpw
        pltpu.sync_copy(idx_hbm.at[pl.ds(base, bpw)], idx_v)
        pltpu.async_copy(table_hbm.at[idx_v], rows_v, sem).wait()
        pltpu.sync_copy(rows_v, out_hbm.at[pl.ds(base, bpw)])

    return _sc_gather_kernel


def _gather(table, idx):
    return _make_sc_gather()(table, idx)


def _finale_body(x_ref, r2_ref, q2_ref, w3_ref, n1_ref, n2_ref,
                 i0_ref, i1_ref, i2_ref, quant_ref, loss_ref, ipack_ref):
    r3 = r2_ref[...] - q2_ref[...]
    q3 = w3_ref[...]
    d3 = q3 - r3
    quant_ref[...] = (x_ref[...] - r3) + q3
    n3 = jnp.sum(r3 * r3, axis=1, keepdims=True)
    nq = jnp.sum(d3 * d3, axis=1, keepdims=True)
    c = jnp.float32((1.0 + MU) / D)
    loss_ref[...] = ((n1_ref[...] * c + n2_ref[...] * c) + n3 * c) + nq * c
    zc = jnp.zeros((BT, 1), jnp.int32)
    ipack_ref[...] = jnp.concatenate(
        [i0_ref[...], zc, i1_ref[...], zc, i2_ref[...], zc, zc, zc], axis=1)


_finale = pl.pallas_call(
    _finale_body,
    grid=(NB,),
    in_specs=[
        pl.BlockSpec((BT, D), lambda b: (b, _Z)),
        pl.BlockSpec((BT, D), lambda b: (b, _Z)),
        pl.BlockSpec((BT, D), lambda b: (b, _Z)),
        pl.BlockSpec((1, D), lambda b: (_Z, _Z)),
        pl.BlockSpec((BT, 1), lambda b: (b, _Z)),
        pl.BlockSpec((BT, 1), lambda b: (b, _Z)),
        pl.BlockSpec((BT, 1), lambda b: (b, _Z)),
        pl.BlockSpec((BT, 1), lambda b: (b, _Z)),
        pl.BlockSpec((BT, 1), lambda b: (b, _Z)),
    ],
    out_specs=[
        pl.BlockSpec((BT, D), lambda b: (b, _Z)),
        pl.BlockSpec((BT, 1), lambda b: (b, _Z)),
        pl.BlockSpec((BT, 8), lambda b: (b, _Z)),
    ],
    out_shape=[
        jax.ShapeDtypeStruct((B, D), jnp.float32),
        jax.ShapeDtypeStruct((B, 1), jnp.float32),
        jax.ShapeDtypeStruct((B, 8), jnp.int32),
    ],
)


def kernel(x, codebooks):
    x = x.astype(jnp.float32)
    codebooks = codebooks.astype(jnp.float32)
    cb = codebooks.reshape(4 * K, D)
    w3row = codebooks[3, 0:1, :]

    idx0, g0, r0, _ = _level0(x, cb)
    q0 = _gather(cb, g0.reshape(B))
    idx1, g1, r1, n1 = _level1(r0, q0, cb)
    q1 = _gather(cb, g1.reshape(B))
    idx2, g2, r2, n2 = _level2(r1, q1, cb)
    q2 = _gather(cb, g2.reshape(B))
    quant, loss, ipack = _finale(x, r2, q2, w3row, n1, n2, idx0, idx1, idx2)

    indices = lax.bitcast_convert_type(ipack.reshape(B, 4, 2), jnp.int64)
    return quant, indices, loss.reshape(B)


# speedup vs baseline: 1.4696x; 1.0305x over previous
import functools

import numpy as np
import jax
import jax.numpy as jnp
from jax import lax
from jax.experimental import pallas as pl
from jax.experimental.pallas import tpu as pltpu
from jax.experimental.pallas import tpu_sc as plsc

K = 8192
D = 256
B = 4096
MU = 0.25
_Z = np.int32(0)

BT = 2048
KT = 4096
NB = B // BT
NK = K // KT


def _make_level(lvl, has_q):
    def body(*refs):
        if has_q:
            (rprev_ref, qprev_ref, cb_ref, idx_ref, gidx_ref, r_ref, n_ref,
             bval_ref, bidx_ref) = refs
        else:
            (rprev_ref, cb_ref, idx_ref, gidx_ref, r_ref, n_ref,
             bval_ref, bidx_ref) = refs
        k = pl.program_id(1)
        if has_q:
            r = rprev_ref[...] - qprev_ref[...]
        else:
            r = rprev_ref[...]
        rn = jnp.sum(r * r, axis=1, keepdims=True)

        @pl.when(k == 0)
        def _init():
            r_ref[...] = r
            n_ref[...] = rn
            bval_ref[...] = jnp.full((BT, 1), jnp.inf, jnp.float32)
            bidx_ref[...] = jnp.zeros((BT, 1), jnp.int32)

        w = cb_ref[...]
        wnt = jnp.sum(w * w, axis=1).reshape(1, KT)
        mm2 = lax.dot_general(r * (-2.0), w, (((1,), (1,)), ((), ())))
        d2 = (rn + wnt) + mm2

        tmin = jnp.min(d2, axis=1, keepdims=True)
        kio = lax.broadcasted_iota(jnp.int32, (BT, KT), 1)
        cand = jnp.where(d2 == tmin, kio, K)
        targ = jnp.min(cand, axis=1, keepdims=True) + k * KT

        upd = tmin < bval_ref[...]
        bidx_ref[...] = jnp.where(upd, targ, bidx_ref[...])
        bval_ref[...] = jnp.where(upd, tmin, bval_ref[...])

        @pl.when(k == pl.num_programs(1) - 1)
        def _fin():
            best = bidx_ref[...]
            idx_ref[...] = best
            gidx_ref[...] = (best + np.int32(lvl * K)).reshape(1, BT)

    base = np.int32(lvl * NK)
    in_specs = [pl.BlockSpec((BT, D), lambda b, k: (b, _Z))]
    if has_q:
        in_specs.append(pl.BlockSpec((BT, D), lambda b, k: (b, _Z)))
    in_specs.append(pl.BlockSpec((KT, D), lambda b, k: (base + k, _Z)))
    return pl.pallas_call(
        body,
        grid=(NB, NK),
        in_specs=in_specs,
        out_specs=[
            pl.BlockSpec((BT, 1), lambda b, k: (b, _Z)),
            pl.BlockSpec((1, BT), lambda b, k: (_Z, b)),
            pl.BlockSpec((BT, D), lambda b, k: (b, _Z)),
            pl.BlockSpec((BT, 1), lambda b, k: (b, _Z)),
        ],
        out_shape=[
            jax.ShapeDtypeStruct((B, 1), jnp.int32),
            jax.ShapeDtypeStruct((1, B), jnp.int32),
            jax.ShapeDtypeStruct((B, D), jnp.float32),
            jax.ShapeDtypeStruct((B, 1), jnp.float32),
        ],
        scratch_shapes=[
            pltpu.VMEM((BT, 1), jnp.float32),
            pltpu.VMEM((BT, 1), jnp.int32),
        ],
    )


_level0 = _make_level(0, has_q=False)
_level1 = _make_level(1, has_q=True)
_level2 = _make_level(2, has_q=True)


@functools.lru_cache(maxsize=1)
def _make_sc_gather():
    info = plsc.get_sparse_core_info()
    nc = info.num_cores
    nw = nc * info.num_subcores
    bpw = B // nw

    @functools.partial(
        pl.kernel,
        out_type=jax.ShapeDtypeStruct((B, D), jnp.float32),
        mesh=plsc.VectorSubcoreMesh(core_axis_name="c", subcore_axis_name="s"),
        scratch_types=[
            pltpu.VMEM((bpw,), jnp.int32),
            pltpu.VMEM((bpw, D), jnp.float32),
            pltpu.SemaphoreType.DMA,
        ],
    )
    def _sc_gather_kernel(table_hbm, idx_hbm, out_hbm, idx_v, rows_v, sem):
        wid = lax.axis_index("s") * nc + lax.axis_index("c")
        base = wid * bpw
        pltpu.sync_copy(idx_hbm.at[pl.ds(base, bpw)], idx_v)
        pltpu.async_copy(table_hbm.at[idx_v], rows_v, sem).wait()
        pltpu.sync_copy(rows_v, out_hbm.at[pl.ds(base, bpw)])

    return _sc_gather_kernel


def _gather(table, idx):
    return _make_sc_gather()(table, idx)


def _finale_body(x_ref, r2_ref, q2_ref, w3_ref, n1_ref, n2_ref,
                 i0_ref, i1_ref, i2_ref, quant_ref, loss_ref, ipack_ref):
    r3 = r2_ref[...] - q2_ref[...]
    q3 = w3_ref[...]
    d3 = q3 - r3
    quant_ref[...] = (x_ref[...] - r3) + q3
    n3 = jnp.sum(r3 * r3, axis=1, keepdims=True)
    nq = jnp.sum(d3 * d3, axis=1, keepdims=True)
    c = jnp.float32((1.0 + MU) / D)
    loss_ref[...] = ((n1_ref[...] * c + n2_ref[...] * c) + n3 * c) + nq * c
    zc = jnp.zeros((BT, 1), jnp.int32)
    ipack_ref[...] = jnp.concatenate(
        [i0_ref[...], zc, i1_ref[...], zc, i2_ref[...], zc, zc, zc], axis=1)


_finale = pl.pallas_call(
    _finale_body,
    grid=(NB,),
    in_specs=[
        pl.BlockSpec((BT, D), lambda b: (b, _Z)),
        pl.BlockSpec((BT, D), lambda b: (b, _Z)),
        pl.BlockSpec((BT, D), lambda b: (b, _Z)),
        pl.BlockSpec((1, D), lambda b: (_Z, _Z)),
        pl.BlockSpec((BT, 1), lambda b: (b, _Z)),
        pl.BlockSpec((BT, 1), lambda b: (b, _Z)),
        pl.BlockSpec((BT, 1), lambda b: (b, _Z)),
        pl.BlockSpec((BT, 1), lambda b: (b, _Z)),
        pl.BlockSpec((BT, 1), lambda b: (b, _Z)),
    ],
    out_specs=[
        pl.BlockSpec((BT, D), lambda b: (b, _Z)),
        pl.BlockSpec((BT, 1), lambda b: (b, _Z)),
        pl.BlockSpec((BT, 8), lambda b: (b, _Z)),
    ],
    out_shape=[
        jax.ShapeDtypeStruct((B, D), jnp.float32),
        jax.ShapeDtypeStruct((B, 1), jnp.float32),
        jax.ShapeDtypeStruct((B, 8), jnp.int32),
    ],
)


def kernel(x, codebooks):
    x = x.astype(jnp.float32)
    codebooks = codebooks.astype(jnp.float32)
    cb = codebooks.reshape(4 * K, D)
    w3row = codebooks[3, 0:1, :]

    idx0, g0, r0, _ = _level0(x, cb)
    q0 = _gather(cb, g0.reshape(B))
    idx1, g1, r1, n1 = _level1(r0, q0, cb)
    q1 = _gather(cb, g1.reshape(B))
    idx2, g2, r2, n2 = _level2(r1, q1, cb)
    q2 = _gather(cb, g2.reshape(B))
    quant, loss, ipack = _finale(x, r2, q2, w3row, n1, n2, idx0, idx1, idx2)

    indices = lax.bitcast_convert_type(ipack.reshape(B, 4, 2), jnp.int64)
    return quant, indices, loss.reshape(B)
